# Initial kernel scaffold; baseline (speedup 1.0000x reference)
import jax
import jax.numpy as jnp
from jax.experimental import pallas as pl


def kernel(h, W_enc, b_enc, W_dec, b_dec):
    raise NotImplementedError("write your pallas kernel here")


# speedup vs baseline: 9.4443x; 9.4443x over previous
import jax
import jax.numpy as jnp
from jax.experimental import pallas as pl

K_TOP = 64


def _encode_body(h_ref, w_ref, b_ref, out_ref):
    acc = jax.lax.dot_general(
        h_ref[...].astype(jnp.bfloat16), w_ref[...].astype(jnp.bfloat16),
        dimension_numbers=(((1,), (1,)), ((), ())),
        preferred_element_type=jnp.float32,
    )
    out_ref[...] = acc + b_ref[...]


def _topk_mask_body(pre_ref, z_ref):
    x = pre_ref[...]
    y = jax.lax.bitcast_convert_type(x, jnp.int32)
    skey = y ^ jax.lax.shift_right_arithmetic(y, 31).__and__(jnp.int32(0x7FFFFFFF))

    def body(i, t):
        cand = t + jnp.left_shift(jnp.int32(1), 31 - i)
        cnt = jnp.sum((skey >= cand).astype(jnp.int32), axis=1, keepdims=True)
        return jnp.where(cnt >= K_TOP, cand, t)

    t0 = jnp.full((x.shape[0], 1), jnp.int32(-(2 ** 31)), jnp.int32)
    t = jax.lax.fori_loop(0, 32, body, t0)
    mask = skey >= t
    z_ref[...] = jnp.where(mask, jnp.maximum(x, 0.0), 0.0)


def _decode_body(z_ref, wd_ref, b_ref, out_ref):
    @pl.when(pl.program_id(1) == 0)
    def _():
        out_ref[...] = jnp.broadcast_to(b_ref[...], out_ref.shape)

    out_ref[...] += jax.lax.dot_general(
        z_ref[...].astype(jnp.bfloat16), wd_ref[...].astype(jnp.bfloat16),
        dimension_numbers=(((1,), (1,)), ((), ())),
        preferred_element_type=jnp.float32,
    )


def kernel(h, W_enc, b_enc, W_dec, b_dec):
    n, d_model = h.shape
    d_sae = W_enc.shape[0]

    bm = min(512, n)
    bn = min(2048, d_sae)
    b2 = b_enc.reshape(1, d_sae)
    pre = pl.pallas_call(
        _encode_body,
        grid=(d_sae // bn, n // bm),
        in_specs=[
            pl.BlockSpec((bm, d_model), lambda c, r: (r, 0)),
            pl.BlockSpec((bn, d_model), lambda c, r: (c, 0)),
            pl.BlockSpec((1, bn), lambda c, r: (0, c)),
        ],
        out_specs=pl.BlockSpec((bm, bn), lambda c, r: (r, c)),
        out_shape=jax.ShapeDtypeStruct((n, d_sae), jnp.float32),
    )(h, W_enc, b2)

    br = min(64, n)
    z = pl.pallas_call(
        _topk_mask_body,
        grid=(n // br,),
        in_specs=[pl.BlockSpec((br, d_sae), lambda r: (r, 0))],
        out_specs=pl.BlockSpec((br, d_sae), lambda r: (r, 0)),
        out_shape=jax.ShapeDtypeStruct((n, d_sae), jnp.float32),
    )(pre)

    bm2 = min(512, n)
    bk = min(2048, d_sae)
    b3 = b_dec.reshape(1, d_model)
    h_hat = pl.pallas_call(
        _decode_body,
        grid=(n // bm2, d_sae // bk),
        in_specs=[
            pl.BlockSpec((bm2, bk), lambda r, k: (r, k)),
            pl.BlockSpec((d_model, bk), lambda r, k: (0, k)),
            pl.BlockSpec((1, d_model), lambda r, k: (0, 0)),
        ],
        out_specs=pl.BlockSpec((bm2, d_model), lambda r, k: (r, 0)),
        out_shape=jax.ShapeDtypeStruct((n, d_model), jnp.float32),
    )(z, W_dec, b3)

    return (h_hat, z)


# speedup vs baseline: 11.2701x; 1.1933x over previous
import jax
import jax.numpy as jnp
from jax.experimental import pallas as pl

K_TOP = 64


def _encode_body(h_ref, w_ref, b_ref, out_ref):
    acc = jax.lax.dot_general(
        h_ref[...].astype(jnp.bfloat16), w_ref[...].astype(jnp.bfloat16),
        dimension_numbers=(((1,), (1,)), ((), ())),
        preferred_element_type=jnp.float32,
    )
    out_ref[...] = acc + b_ref[...]


def _topk_mask_body(pre_ref, z_ref):
    x = pre_ref[...]
    y = jax.lax.bitcast_convert_type(x, jnp.int32)
    skey = y ^ jax.lax.shift_right_arithmetic(y, 31).__and__(jnp.int32(0x7FFFFFFF))

    def cond(state):
        i, _, cnt_t = state
        return jnp.logical_and(i < 32, jnp.logical_not(jnp.all(cnt_t == K_TOP)))

    def body(state):
        i, t, cnt_t = state
        cand = t + jnp.left_shift(jnp.int32(1), 31 - i)
        cnt = jnp.sum((skey >= cand).astype(jnp.int32), axis=1, keepdims=True)
        take = cnt >= K_TOP
        return (i + 1, jnp.where(take, cand, t), jnp.where(take, cnt, cnt_t))

    t0 = jnp.full((x.shape[0], 1), jnp.int32(-(2 ** 31)), jnp.int32)
    c0 = jnp.full((x.shape[0], 1), jnp.int32(x.shape[1]), jnp.int32)
    _, t, _ = jax.lax.while_loop(cond, body, (jnp.int32(0), t0, c0))
    mask = skey >= t
    z_ref[...] = jnp.where(mask, jnp.maximum(x, 0.0), 0.0)


def _decode_body(z_ref, wd_ref, b_ref, out_ref):
    @pl.when(pl.program_id(1) == 0)
    def _():
        out_ref[...] = jnp.broadcast_to(b_ref[...], out_ref.shape)

    out_ref[...] += jax.lax.dot_general(
        z_ref[...].astype(jnp.bfloat16), wd_ref[...].astype(jnp.bfloat16),
        dimension_numbers=(((1,), (1,)), ((), ())),
        preferred_element_type=jnp.float32,
    )


def kernel(h, W_enc, b_enc, W_dec, b_dec):
    n, d_model = h.shape
    d_sae = W_enc.shape[0]

    bm = min(512, n)
    bn = min(2048, d_sae)
    b2 = b_enc.reshape(1, d_sae)
    pre = pl.pallas_call(
        _encode_body,
        grid=(d_sae // bn, n // bm),
        in_specs=[
            pl.BlockSpec((bm, d_model), lambda c, r: (r, 0)),
            pl.BlockSpec((bn, d_model), lambda c, r: (c, 0)),
            pl.BlockSpec((1, bn), lambda c, r: (0, c)),
        ],
        out_specs=pl.BlockSpec((bm, bn), lambda c, r: (r, c)),
        out_shape=jax.ShapeDtypeStruct((n, d_sae), jnp.float32),
    )(h, W_enc, b2)

    br = min(64, n)
    z = pl.pallas_call(
        _topk_mask_body,
        grid=(n // br,),
        in_specs=[pl.BlockSpec((br, d_sae), lambda r: (r, 0))],
        out_specs=pl.BlockSpec((br, d_sae), lambda r: (r, 0)),
        out_shape=jax.ShapeDtypeStruct((n, d_sae), jnp.float32),
    )(pre)

    bm2 = min(1024, n)
    bk = min(1024, d_sae)
    b3 = b_dec.reshape(1, d_model)
    h_hat = pl.pallas_call(
        _decode_body,
        grid=(n // bm2, d_sae // bk),
        in_specs=[
            pl.BlockSpec((bm2, bk), lambda r, k: (r, k)),
            pl.BlockSpec((d_model, bk), lambda r, k: (0, k)),
            pl.BlockSpec((1, d_model), lambda r, k: (0, 0)),
        ],
        out_specs=pl.BlockSpec((bm2, d_model), lambda r, k: (r, 0)),
        out_shape=jax.ShapeDtypeStruct((n, d_model), jnp.float32),
    )(z, W_dec, b3)

    return (h_hat, z)
